# baseline (speedup 1.0000x reference)
import jax
import jax.numpy as jnp
from jax import lax
from jax.experimental import pallas as pl
from jax.experimental.pallas import tpu as pltpu
from jax.experimental.pallas import tpu_sc as plsc

BATCH, SEQ, EMBED = 1024, 200, 64
N = BATCH * SEQ
NC, NS = 2, 16
NW = NC * NS
PER_W = N // NW
CHUNK = 128
NCHUNK = PER_W // CHUNK


def _gather_body(idx_hbm, table_hbm, out_hbm, idx_v, rows_v, gsem):
    c = lax.axis_index("c")
    s = lax.axis_index("s")
    wid = s * NC + c
    base = wid * PER_W
    pltpu.sync_copy(idx_hbm.at[wid], idx_v)

    def body(j, carry):
        pltpu.async_copy(table_hbm.at[idx_v.at[j]], rows_v, gsem).wait()
        pltpu.sync_copy(rows_v, out_hbm.at[pl.ds(base + j * CHUNK, CHUNK)])
        return carry

    lax.fori_loop(0, NCHUNK, body, 0)


def kernel(input_ids, token_embedding):
    ids = input_ids.astype(jnp.int32).reshape(NW, NCHUNK, CHUNK)
    mesh = plsc.VectorSubcoreMesh(core_axis_name="c", subcore_axis_name="s")
    out = pl.kernel(
        _gather_body,
        out_type=jax.ShapeDtypeStruct((N, EMBED), jnp.float32),
        mesh=mesh,
        scratch_types=[
            pltpu.VMEM((NCHUNK, CHUNK), jnp.int32),
            pltpu.VMEM((CHUNK, EMBED), jnp.float32),
            pltpu.SemaphoreType.DMA,
        ],
        compiler_params=pltpu.CompilerParams(use_tc_tiling_on_sc=False),
    )(ids, token_embedding)
    return out.reshape(BATCH, SEQ, EMBED)


# speedup vs baseline: 1.0492x; 1.0492x over previous
import jax
import jax.numpy as jnp
from jax import lax
from jax.experimental import pallas as pl
from jax.experimental.pallas import tpu as pltpu
from jax.experimental.pallas import tpu_sc as plsc

BATCH, SEQ, EMBED = 1024, 200, 64
N = BATCH * SEQ
NC, NS = 2, 16
NW = NC * NS
PER_W = N // NW
CHUNK = 128
NCHUNK = PER_W // CHUNK
NBUF = 5
NGROUP = NCHUNK // NBUF


def _gather_body(idx_hbm, table_hbm, out_hbm, idx_v, rows_v, *sems):
    gsems, osems = sems[:NBUF], sems[NBUF:]
    c = lax.axis_index("c")
    s = lax.axis_index("s")
    wid = s * NC + c
    base = wid * PER_W
    pltpu.sync_copy(idx_hbm.at[wid], idx_v)

    def group(g, carry):
        j0 = g * NBUF
        gathers = []
        for b in range(NBUF):
            @pl.when(g > 0)
            def _(b=b):
                pltpu.make_async_copy(
                    rows_v.at[b], out_hbm.at[pl.ds(base, CHUNK)], osems[b]
                ).wait()
            gathers.append(pltpu.async_copy(
                table_hbm.at[idx_v.at[j0 + b]], rows_v.at[b], gsems[b]))
        for b in range(NBUF):
            gathers[b].wait()
            pltpu.async_copy(
                rows_v.at[b],
                out_hbm.at[pl.ds(base + (j0 + b) * CHUNK, CHUNK)],
                osems[b])
        return carry

    lax.fori_loop(0, NGROUP, group, 0)
    for b in range(NBUF):
        pltpu.make_async_copy(
            rows_v.at[b], out_hbm.at[pl.ds(base, CHUNK)], osems[b]).wait()


def kernel(input_ids, token_embedding):
    ids = input_ids.astype(jnp.int32).reshape(NW, NCHUNK, CHUNK)
    mesh = plsc.VectorSubcoreMesh(core_axis_name="c", subcore_axis_name="s")
    out = pl.kernel(
        _gather_body,
        out_type=jax.ShapeDtypeStruct((N, EMBED), jnp.float32),
        mesh=mesh,
        scratch_types=[
            pltpu.VMEM((NCHUNK, CHUNK), jnp.int32),
            pltpu.VMEM((NBUF, CHUNK, EMBED), jnp.float32),
        ] + [pltpu.SemaphoreType.DMA] * (2 * NBUF),
        compiler_params=pltpu.CompilerParams(use_tc_tiling_on_sc=False),
    )(ids, token_embedding)
    return out.reshape(BATCH, SEQ, EMBED)


# speedup vs baseline: 1.3406x; 1.2777x over previous
import functools

import jax
import jax.numpy as jnp
from jax import lax
from jax.experimental import pallas as pl
from jax.experimental.pallas import tpu as pltpu
from jax.experimental.pallas import tpu_sc as plsc

BATCH, SEQ, EMBED = 1024, 200, 64
VOCAB = 1000000
PAD = 128
N = BATCH * SEQ
NC, NS = 2, 16
NW = NC * NS
PER_W = N // NW
CHUNK = 128
NCHUNK = PER_W // CHUNK
NBUF = 5
NGROUP = NCHUNK // NBUF
TR_W = 2048


def _transpose_body(x_ref, o_ref):
    o_ref[:, :EMBED] = x_ref[...].T


def _relayout_table(tab_t):
    return pl.pallas_call(
        _transpose_body,
        grid=(pl.cdiv(VOCAB, TR_W),),
        in_specs=[pl.BlockSpec((EMBED, TR_W), lambda i: (0, i))],
        out_specs=pl.BlockSpec((TR_W, PAD), lambda i: (i, 0)),
        out_shape=jax.ShapeDtypeStruct((VOCAB, PAD), jnp.float32),
    )(tab_t)


def _gather_body(idx_hbm, table_hbm, out_hbm, idx_v, rows_v, *sems):
    gsems, osems = sems[:NBUF], sems[NBUF:]
    c = lax.axis_index("c")
    s = lax.axis_index("s")
    wid = s * NC + c
    base = wid * PER_W
    pltpu.sync_copy(idx_hbm.at[wid], idx_v)

    def group(g, carry):
        j0 = g * NBUF
        gathers = []
        for b in range(NBUF):
            @pl.when(g > 0)
            def _(b=b):
                pltpu.make_async_copy(
                    rows_v.at[b], out_hbm.at[pl.ds(base, CHUNK)], osems[b]
                ).wait()
            gathers.append(pltpu.async_copy(
                table_hbm.at[idx_v.at[j0 + b]], rows_v.at[b], gsems[b]))
        for b in range(NBUF):
            gathers[b].wait()
            pltpu.async_copy(
                rows_v.at[b],
                out_hbm.at[pl.ds(base + (j0 + b) * CHUNK, CHUNK)],
                osems[b])
        return carry

    lax.fori_loop(0, NGROUP, group, 0)
    for b in range(NBUF):
        pltpu.make_async_copy(
            rows_v.at[b], out_hbm.at[pl.ds(base, CHUNK)], osems[b]).wait()


def kernel(input_ids, token_embedding):
    scratch = _relayout_table(token_embedding.T)
    ids = input_ids.astype(jnp.int32).reshape(NW, NCHUNK, CHUNK)
    mesh = plsc.VectorSubcoreMesh(core_axis_name="c", subcore_axis_name="s")
    out = pl.kernel(
        _gather_body,
        out_type=jax.ShapeDtypeStruct((N, PAD), jnp.float32),
        mesh=mesh,
        scratch_types=[
            pltpu.VMEM((NCHUNK, CHUNK), jnp.int32),
            pltpu.VMEM((NBUF, CHUNK, PAD), jnp.float32),
        ] + [pltpu.SemaphoreType.DMA] * (2 * NBUF),
        compiler_params=pltpu.CompilerParams(use_tc_tiling_on_sc=False),
    )(ids, scratch)
    return out[:, :EMBED].reshape(BATCH, SEQ, EMBED)


# speedup vs baseline: 1.9927x; 1.4865x over previous
import functools

import jax
import jax.numpy as jnp
from jax import lax
from jax.experimental import pallas as pl
from jax.experimental.pallas import tpu as pltpu
from jax.experimental.pallas import tpu_sc as plsc

BATCH, SEQ, EMBED = 1024, 200, 64
VOCAB = 1000000
PAD = 128
N = BATCH * SEQ
NC, NS = 2, 16
NW = NC * NS
PER_W = N // NW
CHUNK = 128
NCHUNK = PER_W // CHUNK
NBUF = 5
NGROUP = NCHUNK // NBUF
TR_W = 8192


def _transpose_body(x_ref, o_ref):
    o_ref[:, :EMBED] = x_ref[...].T


def _relayout_table(tab_t):
    return pl.pallas_call(
        _transpose_body,
        grid=(pl.cdiv(VOCAB, TR_W),),
        in_specs=[pl.BlockSpec((EMBED, TR_W), lambda i: (0, i))],
        out_specs=pl.BlockSpec((TR_W, PAD), lambda i: (i, 0)),
        out_shape=jax.ShapeDtypeStruct((VOCAB, PAD), jnp.float32),
    )(tab_t)


def _gather_body(idx_hbm, table_hbm, out_hbm, idx_v, rows_v, *sems):
    gsems, osems = sems[:NBUF], sems[NBUF:]
    c = lax.axis_index("c")
    s = lax.axis_index("s")
    wid = s * NC + c
    base = wid * PER_W
    pltpu.sync_copy(idx_hbm.at[wid], idx_v)

    def group(g, carry):
        j0 = g * NBUF
        gathers = []
        for b in range(NBUF):
            @pl.when(g > 0)
            def _(b=b):
                pltpu.make_async_copy(
                    rows_v.at[b], out_hbm.at[pl.ds(base, CHUNK)], osems[b]
                ).wait()
            gathers.append(pltpu.async_copy(
                table_hbm.at[idx_v.at[j0 + b]], rows_v.at[b], gsems[b]))
        for b in range(NBUF):
            gathers[b].wait()
            pltpu.async_copy(
                rows_v.at[b],
                out_hbm.at[pl.ds(base + (j0 + b) * CHUNK, CHUNK)],
                osems[b])
        return carry

    lax.fori_loop(0, NGROUP, group, 0)
    for b in range(NBUF):
        pltpu.make_async_copy(
            rows_v.at[b], out_hbm.at[pl.ds(base, CHUNK)], osems[b]).wait()


def kernel(input_ids, token_embedding):
    scratch = _relayout_table(token_embedding.T)
    ids = input_ids.astype(jnp.int32).reshape(NW, NCHUNK, CHUNK)
    mesh = plsc.VectorSubcoreMesh(core_axis_name="c", subcore_axis_name="s")
    out = pl.kernel(
        _gather_body,
        out_type=jax.ShapeDtypeStruct((N, PAD), jnp.float32),
        mesh=mesh,
        scratch_types=[
            pltpu.VMEM((NCHUNK, CHUNK), jnp.int32),
            pltpu.VMEM((NBUF, CHUNK, PAD), jnp.float32),
        ] + [pltpu.SemaphoreType.DMA] * (2 * NBUF),
        compiler_params=pltpu.CompilerParams(use_tc_tiling_on_sc=False),
    )(ids, scratch)
    return out[:, :EMBED].reshape(BATCH, SEQ, EMBED)


# speedup vs baseline: 2.0887x; 1.0482x over previous
import functools

import jax
import jax.numpy as jnp
from jax import lax
from jax.experimental import pallas as pl
from jax.experimental.pallas import tpu as pltpu
from jax.experimental.pallas import tpu_sc as plsc

BATCH, SEQ, EMBED = 1024, 200, 64
VOCAB = 1000000
PAD = 128
N = BATCH * SEQ
NC, NS = 2, 16
NW = NC * NS
PER_W = N // NW
CHUNK = 128
NCHUNK = PER_W // CHUNK
NBUF = 5
NGROUP = NCHUNK // NBUF
TR_W = 16384


def _transpose_body(x_ref, o_ref):
    o_ref[:, :EMBED] = x_ref[...].T


def _relayout_table(tab_t):
    return pl.pallas_call(
        _transpose_body,
        grid=(pl.cdiv(VOCAB, TR_W),),
        in_specs=[pl.BlockSpec((EMBED, TR_W), lambda i: (0, i))],
        out_specs=pl.BlockSpec((TR_W, PAD), lambda i: (i, 0)),
        out_shape=jax.ShapeDtypeStruct((VOCAB, PAD), jnp.float32),
    )(tab_t)


def _gather_body(idx_hbm, table_hbm, out_hbm, idx_v, rows_v, *sems):
    gsems, osems = sems[:NBUF], sems[NBUF:]
    c = lax.axis_index("c")
    s = lax.axis_index("s")
    wid = s * NC + c
    base = wid * PER_W
    pltpu.sync_copy(idx_hbm.at[wid], idx_v)

    def group(g, carry):
        j0 = g * NBUF
        gathers = []
        for b in range(NBUF):
            @pl.when(g > 0)
            def _(b=b):
                pltpu.make_async_copy(
                    rows_v.at[b], out_hbm.at[pl.ds(base, CHUNK)], osems[b]
                ).wait()
            gathers.append(pltpu.async_copy(
                table_hbm.at[idx_v.at[j0 + b]], rows_v.at[b], gsems[b]))
        for b in range(NBUF):
            gathers[b].wait()
            pltpu.async_copy(
                rows_v.at[b],
                out_hbm.at[pl.ds(base + (j0 + b) * CHUNK, CHUNK)],
                osems[b])
        return carry

    lax.fori_loop(0, NGROUP, group, 0)
    for b in range(NBUF):
        pltpu.make_async_copy(
            rows_v.at[b], out_hbm.at[pl.ds(base, CHUNK)], osems[b]).wait()


def kernel(input_ids, token_embedding):
    scratch = _relayout_table(token_embedding.T)
    ids = input_ids.astype(jnp.int32).reshape(NW, NCHUNK, CHUNK)
    mesh = plsc.VectorSubcoreMesh(core_axis_name="c", subcore_axis_name="s")
    out = pl.kernel(
        _gather_body,
        out_type=jax.ShapeDtypeStruct((N, PAD), jnp.float32),
        mesh=mesh,
        scratch_types=[
            pltpu.VMEM((NCHUNK, CHUNK), jnp.int32),
            pltpu.VMEM((NBUF, CHUNK, PAD), jnp.float32),
        ] + [pltpu.SemaphoreType.DMA] * (2 * NBUF),
        compiler_params=pltpu.CompilerParams(use_tc_tiling_on_sc=False),
    )(ids, scratch)
    return out[:, :EMBED].reshape(BATCH, SEQ, EMBED)


# speedup vs baseline: 2.1259x; 1.0178x over previous
import functools

import jax
import jax.numpy as jnp
from jax import lax
from jax.experimental import pallas as pl
from jax.experimental.pallas import tpu as pltpu
from jax.experimental.pallas import tpu_sc as plsc

BATCH, SEQ, EMBED = 1024, 200, 64
VOCAB = 1000000
PAD = 128
N = BATCH * SEQ
NC, NS = 2, 16
NW = NC * NS
PER_W = N // NW
CHUNK = 128
NCHUNK = PER_W // CHUNK
NBUF = 5
NGROUP = NCHUNK // NBUF
TR_W = 32768


def _transpose_body(x_ref, o_ref):
    o_ref[:, :EMBED] = x_ref[...].T


def _relayout_table(tab_t):
    return pl.pallas_call(
        _transpose_body,
        grid=(pl.cdiv(VOCAB, TR_W),),
        in_specs=[pl.BlockSpec((EMBED, TR_W), lambda i: (0, i))],
        out_specs=pl.BlockSpec((TR_W, PAD), lambda i: (i, 0)),
        out_shape=jax.ShapeDtypeStruct((VOCAB, PAD), jnp.float32),
    )(tab_t)


def _gather_body(idx_hbm, table_hbm, out_hbm, idx_v, rows_v, *sems):
    gsems, osems = sems[:NBUF], sems[NBUF:]
    c = lax.axis_index("c")
    s = lax.axis_index("s")
    wid = s * NC + c
    base = wid * PER_W
    pltpu.sync_copy(idx_hbm.at[wid], idx_v)

    def group(g, carry):
        j0 = g * NBUF
        gathers = []
        for b in range(NBUF):
            @pl.when(g > 0)
            def _(b=b):
                pltpu.make_async_copy(
                    rows_v.at[b], out_hbm.at[pl.ds(base, CHUNK)], osems[b]
                ).wait()
            gathers.append(pltpu.async_copy(
                table_hbm.at[idx_v.at[j0 + b]], rows_v.at[b], gsems[b]))
        for b in range(NBUF):
            gathers[b].wait()
            pltpu.async_copy(
                rows_v.at[b],
                out_hbm.at[pl.ds(base + (j0 + b) * CHUNK, CHUNK)],
                osems[b])
        return carry

    lax.fori_loop(0, NGROUP, group, 0)
    for b in range(NBUF):
        pltpu.make_async_copy(
            rows_v.at[b], out_hbm.at[pl.ds(base, CHUNK)], osems[b]).wait()


def kernel(input_ids, token_embedding):
    scratch = _relayout_table(token_embedding.T)
    ids = input_ids.astype(jnp.int32).reshape(NW, NCHUNK, CHUNK)
    mesh = plsc.VectorSubcoreMesh(core_axis_name="c", subcore_axis_name="s")
    out = pl.kernel(
        _gather_body,
        out_type=jax.ShapeDtypeStruct((N, PAD), jnp.float32),
        mesh=mesh,
        scratch_types=[
            pltpu.VMEM((NCHUNK, CHUNK), jnp.int32),
            pltpu.VMEM((NBUF, CHUNK, PAD), jnp.float32),
        ] + [pltpu.SemaphoreType.DMA] * (2 * NBUF),
        compiler_params=pltpu.CompilerParams(use_tc_tiling_on_sc=False),
    )(ids, scratch)
    return out[:, :EMBED].reshape(BATCH, SEQ, EMBED)
